# Initial kernel scaffold; baseline (speedup 1.0000x reference)
import jax
import jax.numpy as jnp
from jax.experimental import pallas as pl


def kernel(x, edge_index, W1, b1, W2, b2):
    raise NotImplementedError("write your pallas kernel here")


# speedup vs baseline: 52.2409x; 52.2409x over previous
import functools

import jax
import jax.numpy as jnp
from jax import lax
from jax.experimental import pallas as pl
from jax.experimental.pallas import tpu as pltpu
from jax.experimental.pallas import tpu_sc as plsc

NN = 10000
EE = 320000
DIN = 128
DHID = 16
DOUT = 6

NC = 2
NS = 16
NW = NC * NS
NPAD = 10240
TRASH = NPAD - NN

CH = 128
NCH = 80
EPW = CH * NCH
EPAD = NW * EPW
NG = NCH // 2

RP = NPAD // NS

BLK = 2048
GRID = NPAD // BLK

_mesh = plsc.VectorSubcoreMesh(core_axis_name="c", subcore_axis_name="s")


@functools.partial(
    pl.kernel,
    mesh=_mesh,
    out_type=jax.ShapeDtypeStruct((NC, NPAD), jnp.float32),
    scratch_types=[
        pltpu.VMEM((NCH, CH), jnp.int32),
        pltpu.VMEM((CH,), jnp.float32),
        pltpu.VMEM_SHARED((NPAD,), jnp.float32),
    ],
)
def _deg_kernel(dst_hbm, ones_hbm, zeros1_hbm, deg_hbm, didx_v, ones_v, acc_sh):
    cid = lax.axis_index("c")
    sid = lax.axis_index("s")
    wid = sid * NC + cid
    pltpu.sync_copy(dst_hbm.at[wid], didx_v)
    pltpu.sync_copy(ones_hbm, ones_v)
    pltpu.sync_copy(zeros1_hbm.at[pl.ds(sid * RP, RP)],
                    acc_sh.at[pl.ds(sid * RP, RP)])
    plsc.subcore_barrier()

    def body(j, c):
        pltpu.sync_copy(ones_v, acc_sh.at[didx_v.at[j]], add=True)
        return c

    lax.fori_loop(0, NCH, body, 0)
    plsc.subcore_barrier()
    pltpu.sync_copy(acc_sh.at[pl.ds(sid * RP, RP)],
                    deg_hbm.at[cid, pl.ds(sid * RP, RP)])


@functools.partial(
    pl.kernel,
    mesh=_mesh,
    out_type=jax.ShapeDtypeStruct((NC, NPAD, DHID), jnp.float32),
    scratch_types=[
        pltpu.VMEM((NCH, CH), jnp.int32),
        pltpu.VMEM((NCH, CH), jnp.int32),
        pltpu.VMEM((CH, DHID), jnp.float32),
        pltpu.VMEM((CH, DHID), jnp.float32),
        pltpu.VMEM_SHARED((NPAD, DHID), jnp.float32),
        pltpu.VMEM_SHARED((NPAD, DHID), jnp.float32),
        pltpu.SemaphoreType.DMA,
        pltpu.SemaphoreType.DMA,
    ],
)
def _agg_kernel(table_hbm, src_hbm, dst_hbm, zeros_hbm, out_hbm,
                sidx_v, didx_v, buf0, buf1, table_sh, acc_sh, sem0, sem1):
    cid = lax.axis_index("c")
    sid = lax.axis_index("s")
    wid = sid * NC + cid
    pltpu.sync_copy(src_hbm.at[wid], sidx_v)
    pltpu.sync_copy(dst_hbm.at[wid], didx_v)
    pltpu.sync_copy(table_hbm.at[pl.ds(sid * RP, RP)],
                    table_sh.at[pl.ds(sid * RP, RP)])
    pltpu.sync_copy(zeros_hbm.at[pl.ds(sid * RP, RP)],
                    acc_sh.at[pl.ds(sid * RP, RP)])
    plsc.subcore_barrier()

    def body(j, c):
        pltpu.async_copy(table_sh.at[sidx_v.at[j]], buf0, sem0).wait()
        pltpu.sync_copy(buf0, acc_sh.at[didx_v.at[j]], add=True)
        return c

    lax.fori_loop(0, NCH, body, 0)
    plsc.subcore_barrier()
    pltpu.sync_copy(acc_sh.at[pl.ds(sid * RP, RP)],
                    out_hbm.at[cid, pl.ds(sid * RP, RP)])


def _mm_body(x_ref, w_ref, degp_ref, hp_ref, dinvb_ref):
    deg = degp_ref[0, :] + degp_ref[1, :] + 1.0
    dinv = lax.rsqrt(deg)
    h = jnp.dot(x_ref[...], w_ref[...], preferred_element_type=jnp.float32)
    hp_ref[...] = h * dinv[:, None]
    dinvb_ref[...] = jnp.broadcast_to(dinv[:, None], (BLK, DHID))


def _mm_call(x_pad, W1, degp):
    return pl.pallas_call(
        _mm_body,
        grid=(GRID,),
        in_specs=[
            pl.BlockSpec((BLK, DIN), lambda i: (i, 0)),
            pl.BlockSpec((DIN, DHID), lambda i: (0, 0)),
            pl.BlockSpec((NC, BLK), lambda i: (0, i)),
        ],
        out_specs=[
            pl.BlockSpec((BLK, DHID), lambda i: (i, 0)),
            pl.BlockSpec((BLK, DHID), lambda i: (i, 0)),
        ],
        out_shape=[
            jax.ShapeDtypeStruct((NPAD, DHID), jnp.float32),
            jax.ShapeDtypeStruct((NPAD, DHID), jnp.float32),
        ],
    )(x_pad, W1, degp)


def _mid_body(p_ref, hp1_ref, dinvb_ref, b1_ref, hp2_ref):
    s = p_ref[0] + p_ref[1] + hp1_ref[...]
    out1 = dinvb_ref[...] * s + b1_ref[...]
    hp2_ref[...] = jnp.maximum(out1, 0.0) * dinvb_ref[...]


def _mid_call(p, hp1, dinvb, b1):
    return pl.pallas_call(
        _mid_body,
        grid=(GRID,),
        in_specs=[
            pl.BlockSpec((NC, BLK, DHID), lambda i: (0, i, 0)),
            pl.BlockSpec((BLK, DHID), lambda i: (i, 0)),
            pl.BlockSpec((BLK, DHID), lambda i: (i, 0)),
            pl.BlockSpec((1, DHID), lambda i: (0, 0)),
        ],
        out_specs=pl.BlockSpec((BLK, DHID), lambda i: (i, 0)),
        out_shape=jax.ShapeDtypeStruct((NPAD, DHID), jnp.float32),
    )(p, hp1, dinvb, b1)


def _out_body(q_ref, hp2_ref, dinvb_ref, w2_ref, b2_ref, o_ref):
    agg = dinvb_ref[...] * (q_ref[0] + q_ref[1] + hp2_ref[...])
    h2 = jnp.dot(agg, w2_ref[...], preferred_element_type=jnp.float32)
    h2 = h2 + b2_ref[...]
    m = jnp.max(h2, axis=1, keepdims=True)
    e = jnp.exp(h2 - m)
    y = e / jnp.sum(e, axis=1, keepdims=True)
    m2 = jnp.max(y, axis=1, keepdims=True)
    e2 = jnp.exp(y - m2)
    o_ref[...] = (y - m2) - jnp.log(jnp.sum(e2, axis=1, keepdims=True))


def _out_call(q, hp2, dinvb, W2, b2):
    return pl.pallas_call(
        _out_body,
        grid=(GRID,),
        in_specs=[
            pl.BlockSpec((NC, BLK, DHID), lambda i: (0, i, 0)),
            pl.BlockSpec((BLK, DHID), lambda i: (i, 0)),
            pl.BlockSpec((BLK, DHID), lambda i: (i, 0)),
            pl.BlockSpec((DHID, DOUT), lambda i: (0, 0)),
            pl.BlockSpec((1, DOUT), lambda i: (0, 0)),
        ],
        out_specs=pl.BlockSpec((BLK, DOUT), lambda i: (i, 0)),
        out_shape=jax.ShapeDtypeStruct((NPAD, DOUT), jnp.float32),
    )(q, hp2, dinvb, W2, b2)


def kernel(x, edge_index, W1, b1, W2, b2):
    src = edge_index[0].astype(jnp.int32)
    dst = edge_index[1].astype(jnp.int32)
    pad_i = jnp.arange(EPAD - EE, dtype=jnp.int32)
    src3 = jnp.concatenate([src, pad_i % NN]).reshape(NW, NCH, CH)
    dst3 = jnp.concatenate([dst, NN + pad_i % TRASH]).reshape(NW, NCH, CH)
    x_pad = jnp.pad(x, ((0, NPAD - NN), (0, 0)))

    ones_h = jnp.ones((CH,), jnp.float32)
    zeros1 = jnp.zeros((NPAD,), jnp.float32)
    zeros16 = jnp.zeros((NPAD, DHID), jnp.float32)

    degp = _deg_kernel(dst3, ones_h, zeros1)
    hp1, dinvb = _mm_call(x_pad, W1, degp)
    p = _agg_kernel(hp1, src3, dst3, zeros16)
    hp2 = _mid_call(p, hp1, dinvb, b1.reshape(1, DHID))
    q = _agg_kernel(hp2, src3, dst3, zeros16)
    out = _out_call(q, hp2, dinvb, W2, b2.reshape(1, DOUT))
    return out[:NN]
